# Initial kernel scaffold; baseline (speedup 1.0000x reference)
import jax
import jax.numpy as jnp
from jax.experimental import pallas as pl


def kernel(node_xyz, xyz, node_feats, xyz_feats, node_fpfhs, xyz_fpfhs):
    raise NotImplementedError("write your pallas kernel here")


# speedup vs baseline: 23.9146x; 23.9146x over previous
import math

import jax
import jax.numpy as jnp
from jax import lax
from jax.experimental import pallas as pl
from jax.experimental.pallas import tpu as pltpu

B, M, N, D, DF = 2, 16384, 1024, 256, 33
DFP = 64
TOP_K = 20
N_SK_ITERS = 5

TM_A = 1024
TM_B = 2048
TM_C = 512

_NEG = -1e30


def _scores_kernel(xf_ref, nf_ref, xg_ref, ng_ref, sf_ref, sg_ref):
    sf_ref[0] = lax.dot_general(
        xf_ref[0], nf_ref[0], (((1,), (1,)), ((), ())),
        preferred_element_type=jnp.float32) * (1.0 / math.sqrt(D))
    sg_ref[0] = lax.dot_general(
        xg_ref[0], ng_ref[0], (((1,), (1,)), ((), ())),
        preferred_element_type=jnp.float32) * (1.0 / math.sqrt(DF))


def _rowlse(s, v):
    t = s - v
    m = jnp.max(t, axis=-1, keepdims=True)
    return m + jnp.log(jnp.sum(jnp.exp(t - m), axis=-1, keepdims=True))


def _sinkhorn_kernel(sg_ref, sf_ref, vg_ref, vf_ref,
                     vg_s, vf_s, accg, accf):
    it = pl.program_id(1)
    mt = pl.program_id(2)
    nit = pl.num_programs(1)
    nmt = pl.num_programs(2)

    @pl.when(jnp.logical_and(it == 0, mt == 0))
    def _init_v():
        vg_s[...] = jnp.zeros_like(vg_s)
        vf_s[...] = jnp.zeros_like(vf_s)

    @pl.when(mt == 0)
    def _init_acc():
        accg[...] = jnp.zeros_like(accg)
        accf[...] = jnp.zeros_like(accf)

    sg = sg_ref[0]
    sf = sf_ref[0]
    vg = vg_s[...]
    vf = vf_s[...]
    ug = _rowlse(sg, vg)
    uf = _rowlse(sf, vf)
    accg[...] += jnp.sum(jnp.exp(sg - ug - vg), axis=0, keepdims=True)
    accf[...] += jnp.sum(jnp.exp(sf - uf - vf), axis=0, keepdims=True)

    @pl.when(mt == nmt - 1)
    def _update_v():
        vg_s[...] = vg_s[...] + jnp.log(accg[...])
        vf_s[...] = vf_s[...] + jnp.log(accf[...])

    @pl.when(jnp.logical_and(it == nit - 1, mt == nmt - 1))
    def _emit_v():
        vg_ref[0] = vg_s[...]
        vf_ref[0] = vf_s[...]


def _topk_combine_kernel(sg_ref, sf_ref, vg_ref, vf_ref, nf_ref, ng_ref,
                         xf_ref, xg_ref, of_ref, og_ref):
    z = sg_ref[0] + sf_ref[0] - vg_ref[0] - vf_ref[0]
    rowmax = jnp.max(z, axis=-1, keepdims=True)
    zz = z
    t = rowmax
    for _ in range(TOP_K - 1):
        zz = jnp.where(zz >= t, _NEG, zz)
        t = jnp.max(zz, axis=-1, keepdims=True)
    w = jnp.where(z >= t, jnp.exp(z - rowmax), 0.0)
    w = w / jnp.sum(w, axis=-1, keepdims=True)
    pf = lax.dot_general(w, nf_ref[0], (((1,), (0,)), ((), ())),
                         preferred_element_type=jnp.float32)
    pg = lax.dot_general(w, ng_ref[0], (((1,), (0,)), ((), ())),
                         preferred_element_type=jnp.float32)
    of_ref[0] = (pf + xf_ref[0]) * 0.5
    og_ref[0] = (pg + xg_ref[0]) * 0.5


@jax.jit
def _run(node_feats, xyz_feats, node_fpfhs_p, xyz_fpfhs_p):
    f32 = jnp.float32

    sf, sg = pl.pallas_call(
        _scores_kernel,
        grid=(B, M // TM_A),
        in_specs=[
            pl.BlockSpec((1, TM_A, D), lambda b, mt: (b, mt, 0)),
            pl.BlockSpec((1, N, D), lambda b, mt: (b, 0, 0)),
            pl.BlockSpec((1, TM_A, DFP), lambda b, mt: (b, mt, 0)),
            pl.BlockSpec((1, N, DFP), lambda b, mt: (b, 0, 0)),
        ],
        out_specs=[
            pl.BlockSpec((1, TM_A, N), lambda b, mt: (b, mt, 0)),
            pl.BlockSpec((1, TM_A, N), lambda b, mt: (b, mt, 0)),
        ],
        out_shape=[
            jax.ShapeDtypeStruct((B, M, N), f32),
            jax.ShapeDtypeStruct((B, M, N), f32),
        ],
    )(xyz_feats, node_feats, xyz_fpfhs_p, node_fpfhs_p)

    vg, vf = pl.pallas_call(
        _sinkhorn_kernel,
        grid=(B, N_SK_ITERS, M // TM_B),
        in_specs=[
            pl.BlockSpec((1, TM_B, N), lambda b, it, mt: (b, mt, 0)),
            pl.BlockSpec((1, TM_B, N), lambda b, it, mt: (b, mt, 0)),
        ],
        out_specs=[
            pl.BlockSpec((1, 1, N), lambda b, it, mt: (b, 0, 0)),
            pl.BlockSpec((1, 1, N), lambda b, it, mt: (b, 0, 0)),
        ],
        out_shape=[
            jax.ShapeDtypeStruct((B, 1, N), f32),
            jax.ShapeDtypeStruct((B, 1, N), f32),
        ],
        scratch_shapes=[
            pltpu.VMEM((1, N), f32),
            pltpu.VMEM((1, N), f32),
            pltpu.VMEM((1, N), f32),
            pltpu.VMEM((1, N), f32),
        ],
    )(sg, sf)

    of, og = pl.pallas_call(
        _topk_combine_kernel,
        grid=(B, M // TM_C),
        in_specs=[
            pl.BlockSpec((1, TM_C, N), lambda b, mt: (b, mt, 0)),
            pl.BlockSpec((1, TM_C, N), lambda b, mt: (b, mt, 0)),
            pl.BlockSpec((1, 1, N), lambda b, mt: (b, 0, 0)),
            pl.BlockSpec((1, 1, N), lambda b, mt: (b, 0, 0)),
            pl.BlockSpec((1, N, D), lambda b, mt: (b, 0, 0)),
            pl.BlockSpec((1, N, DFP), lambda b, mt: (b, 0, 0)),
            pl.BlockSpec((1, TM_C, D), lambda b, mt: (b, mt, 0)),
            pl.BlockSpec((1, TM_C, DFP), lambda b, mt: (b, mt, 0)),
        ],
        out_specs=[
            pl.BlockSpec((1, TM_C, D), lambda b, mt: (b, mt, 0)),
            pl.BlockSpec((1, TM_C, DFP), lambda b, mt: (b, mt, 0)),
        ],
        out_shape=[
            jax.ShapeDtypeStruct((B, M, D), f32),
            jax.ShapeDtypeStruct((B, M, DFP), f32),
        ],
    )(sg, sf, vg, vf, node_feats, node_fpfhs_p, xyz_feats, xyz_fpfhs_p)

    return of, og


def kernel(node_xyz, xyz, node_feats, xyz_feats, node_fpfhs, xyz_fpfhs):
    del node_xyz, xyz
    node_fpfhs_p = jnp.pad(node_fpfhs, ((0, 0), (0, 0), (0, DFP - DF)))
    xyz_fpfhs_p = jnp.pad(xyz_fpfhs, ((0, 0), (0, 0), (0, DFP - DF)))
    attn_feats, attn_fpfhs_p = _run(node_feats, xyz_feats,
                                    node_fpfhs_p, xyz_fpfhs_p)
    return attn_feats, attn_fpfhs_p[..., :DF]


# speedup vs baseline: 29.0865x; 1.2163x over previous
import math

import jax
import jax.numpy as jnp
from jax import lax
from jax.experimental import pallas as pl
from jax.experimental.pallas import tpu as pltpu

B, M, N, D, DF = 2, 16384, 1024, 256, 33
DFP = 64
TOP_K = 20
N_SK_ITERS = 5

TM_A = 1024
TM_B = 2048
TM_C = 512

_NEG = -1e30


def _scores_kernel(xf_ref, nf_ref, xg_ref, ng_ref, sf_ref, sg_ref,
                   vg_ref, vf_ref, accg, accf):
    mt = pl.program_id(1)
    nmt = pl.num_programs(1)

    @pl.when(mt == 0)
    def _init_acc():
        accg[...] = jnp.zeros_like(accg)
        accf[...] = jnp.zeros_like(accf)

    sf = lax.dot_general(
        xf_ref[0], nf_ref[0], (((1,), (1,)), ((), ())),
        preferred_element_type=jnp.float32) * (1.0 / math.sqrt(D))
    sg = lax.dot_general(
        xg_ref[0], ng_ref[0], (((1,), (1,)), ((), ())),
        preferred_element_type=jnp.float32) * (1.0 / math.sqrt(DF))
    sf_ref[0] = sf
    sg_ref[0] = sg
    for s, acc in ((sf, accf), (sg, accg)):
        m = jnp.max(s, axis=-1, keepdims=True)
        e = jnp.exp(s - m)
        rs = jnp.sum(e, axis=-1, keepdims=True)
        acc[...] += jnp.sum(e * (1.0 / rs), axis=0, keepdims=True)

    @pl.when(mt == nmt - 1)
    def _emit_v1():
        vg_ref[0] = jnp.log(accg[...])
        vf_ref[0] = jnp.log(accf[...])


def _sinkhorn_kernel(sg_ref, sf_ref, vg1_ref, vf1_ref, vg_ref, vf_ref,
                     vg_s, vf_s, accg, accf):
    it = pl.program_id(1)
    mt = pl.program_id(2)
    nit = pl.num_programs(1)
    nmt = pl.num_programs(2)

    @pl.when(jnp.logical_and(it == 0, mt == 0))
    def _init_v():
        vg_s[...] = vg1_ref[0]
        vf_s[...] = vf1_ref[0]

    @pl.when(mt == 0)
    def _init_acc():
        accg[...] = jnp.zeros_like(accg)
        accf[...] = jnp.zeros_like(accf)

    for s_ref, v_s, acc in ((sg_ref, vg_s, accg), (sf_ref, vf_s, accf)):
        t = s_ref[0] - v_s[...]
        m = jnp.max(t, axis=-1, keepdims=True)
        e = jnp.exp(t - m)
        rs = jnp.sum(e, axis=-1, keepdims=True)
        acc[...] += jnp.sum(e * (1.0 / rs), axis=0, keepdims=True)

    @pl.when(mt == nmt - 1)
    def _update_v():
        vg_s[...] = vg_s[...] + jnp.log(accg[...])
        vf_s[...] = vf_s[...] + jnp.log(accf[...])

    @pl.when(jnp.logical_and(it == nit - 1, mt == nmt - 1))
    def _emit_v():
        vg_ref[0] = vg_s[...]
        vf_ref[0] = vf_s[...]


def _topk_combine_kernel(sg_ref, sf_ref, vg_ref, vf_ref, nf_ref, ng_ref,
                         xf_ref, xg_ref, of_ref, og_ref):
    z = sg_ref[0] + sf_ref[0] - vg_ref[0] - vf_ref[0]
    rowmax = jnp.max(z, axis=-1, keepdims=True)
    zz = z
    t = rowmax
    for _ in range(TOP_K - 1):
        zz = jnp.where(zz >= t, _NEG, zz)
        t = jnp.max(zz, axis=-1, keepdims=True)
    w = jnp.where(z >= t, jnp.exp(z - rowmax), 0.0)
    w = w / jnp.sum(w, axis=-1, keepdims=True)
    pf = lax.dot_general(w, nf_ref[0], (((1,), (0,)), ((), ())),
                         preferred_element_type=jnp.float32)
    pg = lax.dot_general(w, ng_ref[0], (((1,), (0,)), ((), ())),
                         preferred_element_type=jnp.float32)
    of_ref[0] = (pf + xf_ref[0]) * 0.5
    og_ref[0] = (pg + xg_ref[0]) * 0.5


@jax.jit
def _run(node_feats, xyz_feats, node_fpfhs_p, xyz_fpfhs_p):
    f32 = jnp.float32

    sf, sg, vg1, vf1 = pl.pallas_call(
        _scores_kernel,
        grid=(B, M // TM_A),
        in_specs=[
            pl.BlockSpec((1, TM_A, D), lambda b, mt: (b, mt, 0)),
            pl.BlockSpec((1, N, D), lambda b, mt: (b, 0, 0)),
            pl.BlockSpec((1, TM_A, DFP), lambda b, mt: (b, mt, 0)),
            pl.BlockSpec((1, N, DFP), lambda b, mt: (b, 0, 0)),
        ],
        out_specs=[
            pl.BlockSpec((1, TM_A, N), lambda b, mt: (b, mt, 0)),
            pl.BlockSpec((1, TM_A, N), lambda b, mt: (b, mt, 0)),
            pl.BlockSpec((1, 1, N), lambda b, mt: (b, 0, 0)),
            pl.BlockSpec((1, 1, N), lambda b, mt: (b, 0, 0)),
        ],
        out_shape=[
            jax.ShapeDtypeStruct((B, M, N), f32),
            jax.ShapeDtypeStruct((B, M, N), f32),
            jax.ShapeDtypeStruct((B, 1, N), f32),
            jax.ShapeDtypeStruct((B, 1, N), f32),
        ],
        scratch_shapes=[
            pltpu.VMEM((1, N), f32),
            pltpu.VMEM((1, N), f32),
        ],
    )(xyz_feats, node_feats, xyz_fpfhs_p, node_fpfhs_p)

    vg, vf = pl.pallas_call(
        _sinkhorn_kernel,
        grid=(B, N_SK_ITERS - 1, M // TM_B),
        in_specs=[
            pl.BlockSpec((1, TM_B, N), lambda b, it, mt: (b, mt, 0)),
            pl.BlockSpec((1, TM_B, N), lambda b, it, mt: (b, mt, 0)),
            pl.BlockSpec((1, 1, N), lambda b, it, mt: (b, 0, 0)),
            pl.BlockSpec((1, 1, N), lambda b, it, mt: (b, 0, 0)),
        ],
        out_specs=[
            pl.BlockSpec((1, 1, N), lambda b, it, mt: (b, 0, 0)),
            pl.BlockSpec((1, 1, N), lambda b, it, mt: (b, 0, 0)),
        ],
        out_shape=[
            jax.ShapeDtypeStruct((B, 1, N), f32),
            jax.ShapeDtypeStruct((B, 1, N), f32),
        ],
        scratch_shapes=[
            pltpu.VMEM((1, N), f32),
            pltpu.VMEM((1, N), f32),
            pltpu.VMEM((1, N), f32),
            pltpu.VMEM((1, N), f32),
        ],
    )(sg, sf, vg1, vf1)

    of, og = pl.pallas_call(
        _topk_combine_kernel,
        grid=(B, M // TM_C),
        in_specs=[
            pl.BlockSpec((1, TM_C, N), lambda b, mt: (b, mt, 0)),
            pl.BlockSpec((1, TM_C, N), lambda b, mt: (b, mt, 0)),
            pl.BlockSpec((1, 1, N), lambda b, mt: (b, 0, 0)),
            pl.BlockSpec((1, 1, N), lambda b, mt: (b, 0, 0)),
            pl.BlockSpec((1, N, D), lambda b, mt: (b, 0, 0)),
            pl.BlockSpec((1, N, DFP), lambda b, mt: (b, 0, 0)),
            pl.BlockSpec((1, TM_C, D), lambda b, mt: (b, mt, 0)),
            pl.BlockSpec((1, TM_C, DFP), lambda b, mt: (b, mt, 0)),
        ],
        out_specs=[
            pl.BlockSpec((1, TM_C, D), lambda b, mt: (b, mt, 0)),
            pl.BlockSpec((1, TM_C, DFP), lambda b, mt: (b, mt, 0)),
        ],
        out_shape=[
            jax.ShapeDtypeStruct((B, M, D), f32),
            jax.ShapeDtypeStruct((B, M, DFP), f32),
        ],
    )(sg, sf, vg, vf, node_feats, node_fpfhs_p, xyz_feats, xyz_fpfhs_p)

    return of, og


def kernel(node_xyz, xyz, node_feats, xyz_feats, node_fpfhs, xyz_fpfhs):
    del node_xyz, xyz
    node_fpfhs_p = jnp.pad(node_fpfhs, ((0, 0), (0, 0), (0, DFP - DF)))
    xyz_fpfhs_p = jnp.pad(xyz_fpfhs, ((0, 0), (0, 0), (0, DFP - DF)))
    attn_feats, attn_fpfhs_p = _run(node_feats, xyz_feats,
                                    node_fpfhs_p, xyz_fpfhs_p)
    return attn_feats, attn_fpfhs_p[..., :DF]
